# Initial kernel scaffold; baseline (speedup 1.0000x reference)
import jax
import jax.numpy as jnp
from jax.experimental import pallas as pl


def kernel(x, pos, edge_index, batch, labels, W1, a_s1, a_d1, b1, W2, a_s2, a_d2, b2, Wl1, bl1, Wl2, bl2):
    raise NotImplementedError("write your pallas kernel here")


# speedup vs baseline: 11.5453x; 11.5453x over previous
import functools

import jax
import jax.numpy as jnp
from jax import lax
from jax.experimental import pallas as pl
from jax.experimental.pallas import tpu as pltpu
from jax.experimental.pallas import tpu_sc as plsc

N = 10000
D = 256
DH = 128
G = 16
C = 40
NPAD = 10240
NPT = NPAD // 16
E_REAL = 160000 + N
ET = 10752
EPAD = 16 * ET
RCH = 64
GROUPS = 3
GE = ET // GROUPS
NCHG = GE // RCH
NCHT = ET // RCH
RB = 2048
EPS = 1e-16


def _tc1_body(x_ref, w_ref, as_ref, ad_ref, hl_ref, hr_ref, asv_ref, adv_ref):
    h = jnp.dot(x_ref[...], w_ref[...], preferred_element_type=jnp.float32)
    hl_ref[...] = h[:, :DH]
    hr_ref[...] = h[:, DH:]
    asv_ref[...] = jnp.dot(h, as_ref[...], preferred_element_type=jnp.float32)
    adv_ref[...] = jnp.dot(h, ad_ref[...], preferred_element_type=jnp.float32)


def _tc1(xp, w, a_s, a_d):
    grid = NPAD // RB
    return pl.pallas_call(
        _tc1_body,
        grid=(grid,),
        in_specs=[
            pl.BlockSpec((RB, D), lambda i: (i, 0)),
            pl.BlockSpec((D, D), lambda i: (0, 0)),
            pl.BlockSpec((D, 1), lambda i: (0, 0)),
            pl.BlockSpec((D, 1), lambda i: (0, 0)),
        ],
        out_specs=[
            pl.BlockSpec((RB, DH), lambda i: (i, 0)),
            pl.BlockSpec((RB, DH), lambda i: (i, 0)),
            pl.BlockSpec((RB, 1), lambda i: (i, 0)),
            pl.BlockSpec((RB, 1), lambda i: (i, 0)),
        ],
        out_shape=[
            jax.ShapeDtypeStruct((NPAD, DH), jnp.float32),
            jax.ShapeDtypeStruct((NPAD, DH), jnp.float32),
            jax.ShapeDtypeStruct((NPAD, 1), jnp.float32),
            jax.ShapeDtypeStruct((NPAD, 1), jnp.float32),
        ],
    )(xp, w, a_s, a_d)


def _tc2_body(accl_ref, accr_ref, z_ref, w_ref, as_ref, ad_ref, bl_ref, br_ref,
              hl_ref, hr_ref, asv_ref, adv_ref):
    zc = z_ref[...] + EPS
    outl = jnp.maximum(accl_ref[...] / zc + bl_ref[...], 0.0)
    outr = jnp.maximum(accr_ref[...] / zc + br_ref[...], 0.0)
    w = w_ref[...]
    h = (jnp.dot(outl, w[:DH, :], preferred_element_type=jnp.float32)
         + jnp.dot(outr, w[DH:, :], preferred_element_type=jnp.float32))
    hl_ref[...] = h[:, :DH]
    hr_ref[...] = h[:, DH:]
    asv_ref[...] = jnp.dot(h, as_ref[...], preferred_element_type=jnp.float32)
    adv_ref[...] = jnp.dot(h, ad_ref[...], preferred_element_type=jnp.float32)


def _tc2(accl, accr, z, w, a_s, a_d, bl, br):
    grid = NPAD // RB
    return pl.pallas_call(
        _tc2_body,
        grid=(grid,),
        in_specs=[
            pl.BlockSpec((RB, DH), lambda i: (i, 0)),
            pl.BlockSpec((RB, DH), lambda i: (i, 0)),
            pl.BlockSpec((RB, 1), lambda i: (i, 0)),
            pl.BlockSpec((D, D), lambda i: (0, 0)),
            pl.BlockSpec((D, 1), lambda i: (0, 0)),
            pl.BlockSpec((D, 1), lambda i: (0, 0)),
            pl.BlockSpec((1, DH), lambda i: (0, 0)),
            pl.BlockSpec((1, DH), lambda i: (0, 0)),
        ],
        out_specs=[
            pl.BlockSpec((RB, DH), lambda i: (i, 0)),
            pl.BlockSpec((RB, DH), lambda i: (i, 0)),
            pl.BlockSpec((RB, 1), lambda i: (i, 0)),
            pl.BlockSpec((RB, 1), lambda i: (i, 0)),
        ],
        out_shape=[
            jax.ShapeDtypeStruct((NPAD, DH), jnp.float32),
            jax.ShapeDtypeStruct((NPAD, DH), jnp.float32),
            jax.ShapeDtypeStruct((NPAD, 1), jnp.float32),
            jax.ShapeDtypeStruct((NPAD, 1), jnp.float32),
        ],
    )(accl, accr, z, w, a_s, a_d, bl, br)


def _tc3_body(accl_ref, accr_ref, z_ref, batch_ref, bl_ref, br_ref,
              wl1_ref, bl1_ref, wl2_ref, bl2_ref, lab_ref,
              loss_ref, logits_ref, sl_ref, sr_ref, cnt_ref):
    t = pl.program_id(0)
    zc = z_ref[...] + EPS
    outl = jnp.maximum(accl_ref[...] / zc + bl_ref[...], 0.0)
    outr = jnp.maximum(accr_ref[...] / zc + br_ref[...], 0.0)
    oh = (batch_ref[...] == lax.broadcasted_iota(jnp.int32, (RB, G), 1)
          ).astype(jnp.float32)
    dn = (((0,), (0,)), ((), ()))
    sl = lax.dot_general(oh, outl, dn, preferred_element_type=jnp.float32)
    sr = lax.dot_general(oh, outr, dn, preferred_element_type=jnp.float32)
    cn = lax.dot_general(oh, jnp.ones((RB, DH), jnp.float32), dn,
                         preferred_element_type=jnp.float32)

    @pl.when(t == 0)
    def _():
        sl_ref[...] = sl
        sr_ref[...] = sr
        cnt_ref[...] = cn

    @pl.when(t > 0)
    def _():
        sl_ref[...] += sl
        sr_ref[...] += sr
        cnt_ref[...] += cn

    @pl.when(t == pl.num_programs(0) - 1)
    def _():
        cnt = jnp.maximum(cnt_ref[...], 1.0)
        pld = sl_ref[...] / cnt
        prd = sr_ref[...] / cnt
        wl1 = wl1_ref[...]
        hm = jnp.maximum(
            jnp.dot(pld, wl1[:DH, :], preferred_element_type=jnp.float32)
            + jnp.dot(prd, wl1[DH:, :], preferred_element_type=jnp.float32)
            + bl1_ref[...], 0.0)
        logits = jnp.dot(hm, wl2_ref[...],
                         preferred_element_type=jnp.float32) + bl2_ref[...]
        m = jnp.max(logits, axis=1, keepdims=True)
        lse = m + jnp.log(jnp.sum(jnp.exp(logits - m), axis=1, keepdims=True))
        logp = logits - lse
        lab_oh = (lab_ref[...] == lax.broadcasted_iota(jnp.int32, (G, C), 1)
                  ).astype(jnp.float32)
        loss_ref[...] = (-jnp.sum(logp * lab_oh) * (1.0 / G)).reshape(1, 1)
        logits_ref[...] = logits


def _tc3(accl, accr, z, batchp, bl, br, wl1, bl1, wl2, bl2, labels):
    grid = NPAD // RB
    return pl.pallas_call(
        _tc3_body,
        grid=(grid,),
        in_specs=[
            pl.BlockSpec((RB, DH), lambda i: (i, 0)),
            pl.BlockSpec((RB, DH), lambda i: (i, 0)),
            pl.BlockSpec((RB, 1), lambda i: (i, 0)),
            pl.BlockSpec((RB, 1), lambda i: (i, 0)),
            pl.BlockSpec((1, DH), lambda i: (0, 0)),
            pl.BlockSpec((1, DH), lambda i: (0, 0)),
            pl.BlockSpec((D, DH), lambda i: (0, 0)),
            pl.BlockSpec((1, DH), lambda i: (0, 0)),
            pl.BlockSpec((DH, C), lambda i: (0, 0)),
            pl.BlockSpec((1, C), lambda i: (0, 0)),
            pl.BlockSpec((G, 1), lambda i: (0, 0)),
        ],
        out_specs=[
            pl.BlockSpec((1, 1), lambda i: (0, 0)),
            pl.BlockSpec((G, C), lambda i: (0, 0)),
        ],
        out_shape=[
            jax.ShapeDtypeStruct((1, 1), jnp.float32),
            jax.ShapeDtypeStruct((G, C), jnp.float32),
        ],
        scratch_shapes=[
            pltpu.VMEM((G, DH), jnp.float32),
            pltpu.VMEM((G, DH), jnp.float32),
            pltpu.VMEM((G, DH), jnp.float32),
        ],
    )(accl, accr, z, batchp, bl, br, wl1, bl1, wl2, bl2, labels)


def _sc_body(hl_hbm, hr_hbm, asv_hbm, adv_hbm, src_hbm, dstf_hbm, dst2_hbm,
             z_hbm, accl_hbm, accr_hbm,
             src_c, dst_c, dst2, p_c, asl, adl, gbuf, zbuf, z_s, acc_s, sem):
    c = lax.axis_index("c")
    s = lax.axis_index("s")
    base = s * NPT

    pltpu.sync_copy(asv_hbm, asl)
    pltpu.sync_copy(adv_hbm, adl)

    def _zrow(r, carry):
        for k in range(DH // 16):
            gbuf[r, pl.ds(k * 16, 16)] = jnp.zeros((16,), jnp.float32)
        return carry
    lax.fori_loop(0, RCH, _zrow, 0)

    def _zz(i, carry):
        zbuf[pl.ds(i * 16, 16)] = jnp.zeros((16,), jnp.float32)
        return carry
    lax.fori_loop(0, NPT // 16, _zz, 0)

    for i in range(NPT // RCH):
        pltpu.sync_copy(gbuf, acc_s.at[pl.ds(base + i * RCH, RCH)])
    pltpu.sync_copy(zbuf, z_s.at[pl.ds(base, NPT)])
    plsc.subcore_barrier()

    def _run(h_hbm):
        for g in range(GROUPS):
            pltpu.sync_copy(src_hbm.at[s, pl.ds(g * GE, GE)], src_c)
            pltpu.sync_copy(dstf_hbm.at[s, pl.ds(g * GE, GE)], dst_c)
            pltpu.sync_copy(dst2_hbm.at[s, pl.ds(g * NCHG, NCHG)], dst2)

            def _p1(i, carry):
                sv = src_c[pl.ds(i * 16, 16)]
                dv = dst_c[pl.ds(i * 16, 16)]
                e = plsc.load_gather(asl, [sv]) + plsc.load_gather(adl, [dv])
                e = jnp.where(e >= 0.0, e, e * 0.2)
                p_c[pl.ds(i * 16, 16)] = jnp.exp(e)
                return carry
            lax.fori_loop(0, GE // 16, _p1, 0)

            def _chunk(k, carry):
                pltpu.sync_copy(p_c.at[pl.ds(k * RCH, RCH)],
                                z_s.at[dst2.at[k]], add=True)
                idx = src_c.at[pl.ds(k * RCH, RCH)]
                pltpu.async_copy(h_hbm.at[idx], gbuf, sem).wait()

                def _scale16(q, inner):
                    pv16 = p_c[pl.ds(k * RCH + q * 16, 16)]
                    base_r = q * 16
                    for l in range(16):
                        pv = pv16[l]
                        for f in range(DH // 16):
                            sl = pl.ds(f * 16, 16)
                            gbuf[base_r + l, sl] = gbuf[base_r + l, sl] * pv
                    return inner
                lax.fori_loop(0, RCH // 16, _scale16, 0)
                pltpu.sync_copy(gbuf, acc_s.at[dst2.at[k]], add=True)
                return carry
            lax.fori_loop(0, NCHG, _chunk, 0)

    @pl.when(c == 0)
    def _():
        _run(hl_hbm)

    @pl.when(c == 1)
    def _():
        _run(hr_hbm)

    plsc.subcore_barrier()

    @pl.when(c == 0)
    def _():
        pltpu.sync_copy(z_s.at[pl.ds(base, NPT)], z_hbm.at[pl.ds(base, NPT)])

    def _export(acc_hbm):
        for i in range(NPT // RCH):
            sl = pl.ds(base + i * RCH, RCH)
            pltpu.sync_copy(acc_s.at[sl], acc_hbm.at[sl])

    @pl.when(c == 0)
    def _():
        _export(accl_hbm)

    @pl.when(c == 1)
    def _():
        _export(accr_hbm)


@functools.partial(jax.jit, static_argnames=())
def _sc_edge(hl, hr, asv, adv, src_t, dst_t, dst2d):
    mesh = plsc.VectorSubcoreMesh(core_axis_name="c", subcore_axis_name="s",
                                  num_cores=2, num_subcores=16)
    f = pl.kernel(
        _sc_body,
        out_type=[
            jax.ShapeDtypeStruct((NPAD,), jnp.float32),
            jax.ShapeDtypeStruct((NPAD, DH), jnp.float32),
            jax.ShapeDtypeStruct((NPAD, DH), jnp.float32),
        ],
        mesh=mesh,
        compiler_params=pltpu.CompilerParams(needs_layout_passes=False),
        scratch_types=[
            pltpu.VMEM((GE,), jnp.int32),
            pltpu.VMEM((GE,), jnp.int32),
            pltpu.VMEM((NCHG, RCH), jnp.int32),
            pltpu.VMEM((GE,), jnp.float32),
            pltpu.VMEM((NPAD,), jnp.float32),
            pltpu.VMEM((NPAD,), jnp.float32),
            pltpu.VMEM((RCH, DH), jnp.float32),
            pltpu.VMEM((NPT,), jnp.float32),
            pltpu.VMEM_SHARED((NPAD,), jnp.float32),
            pltpu.VMEM_SHARED((NPAD, DH), jnp.float32),
            pltpu.SemaphoreType.DMA,
        ],
    )
    return f(hl, hr, asv, adv, src_t, dst_t, dst2d)


def kernel(x, pos, edge_index, batch, labels, W1, a_s1, a_d1, b1,
           W2, a_s2, a_d2, b2, Wl1, bl1, Wl2, bl2):
    f32 = jnp.float32
    loop = jnp.arange(N, dtype=jnp.int32)
    src = jnp.concatenate([edge_index[0], loop,
                           jnp.zeros((EPAD - E_REAL,), jnp.int32)])
    dst = jnp.concatenate([edge_index[1], loop,
                           jnp.full((EPAD - E_REAL,), N, jnp.int32)])
    src_t = src.reshape(16, ET)
    dst_t = dst.reshape(16, ET)
    dst2d = dst.reshape(16, NCHT, RCH)

    xp = jnp.zeros((NPAD, D), f32)
    xp = xp.at[:N, :3].set(pos).at[:N, 3:].set(x)

    batchp = jnp.concatenate([batch.astype(jnp.int32),
                              jnp.full((NPAD - N,), G, jnp.int32)]
                             ).reshape(NPAD, 1)
    lab2 = labels.astype(jnp.int32).reshape(G, 1)

    hl1, hr1, as1v, ad1v = _tc1(xp, W1, a_s1.reshape(D, 1), a_d1.reshape(D, 1))
    z1, accl1, accr1 = _sc_edge(hl1, hr1, as1v.reshape(NPAD),
                                ad1v.reshape(NPAD), src_t, dst_t, dst2d)
    hl2, hr2, as2v, ad2v = _tc2(accl1, accr1, z1.reshape(NPAD, 1), W2,
                                a_s2.reshape(D, 1), a_d2.reshape(D, 1),
                                b1[:DH].reshape(1, DH), b1[DH:].reshape(1, DH))
    z2, accl2, accr2 = _sc_edge(hl2, hr2, as2v.reshape(NPAD),
                                ad2v.reshape(NPAD), src_t, dst_t, dst2d)
    loss, logits = _tc3(accl2, accr2, z2.reshape(NPAD, 1), batchp,
                        b2[:DH].reshape(1, DH), b2[DH:].reshape(1, DH),
                        Wl1, bl1.reshape(1, DH), Wl2, bl2.reshape(1, C), lab2)
    return (loss.reshape(()), logits)
